# baseline (speedup 1.0000x reference)
import functools

import jax
import jax.numpy as jnp
from jax import lax
from jax.experimental import pallas as pl
from jax.experimental.pallas import tpu as pltpu
from jax.experimental.pallas import tpu_sc as plsc

VOCAB = 5977
INV_SLOTS = 55
EDIM = 32
HDIM = 128
BATCH = 16384

NC = 2
NS = 16
NW = NC * NS

SLOT_PAD = 56
EDIMP = EDIM // 2
CG = 7
NSEG = 4
SEG_B = BATCH // NSEG
CB = 128
CHUNK = CB * 8
BCHUNKS = SEG_B // CB
N_CHUNKS = CG * BCHUNKS // NW
IDX_PER_W = N_CHUNKS * CHUNK


def _gather_body(idx_hbm, emb_hbm, out_hbm, idx_v, r0, r1, g0, g1, w0, w1):
    wid = lax.axis_index("s") * NC + lax.axis_index("c")
    rows = (r0, r1)
    gsem = (g0, g1)
    wsem = (w0, w1)
    pltpu.sync_copy(idx_hbm.at[pl.ds(wid * IDX_PER_W, IDX_PER_W)], idx_v)

    def start_gather(k):
        return pltpu.async_copy(
            emb_hbm.at[idx_v.at[pl.ds(k * CHUNK, CHUNK)]],
            rows[k % 2], gsem[k % 2])

    def start_writebacks(k):
        t = wid * N_CHUNKS + k
        c = t // BCHUNKS
        b0 = (t % BCHUNKS) * CB
        return [
            pltpu.async_copy(
                rows[k % 2].at[pl.ds(j * CB, CB), :],
                out_hbm.at[c, pl.ds(b0, CB), pl.ds(EDIMP * j, EDIMP)],
                wsem[k % 2])
            for j in range(8)
        ]

    pend_w = {}
    gh = start_gather(0)
    for k in range(N_CHUNKS):
        gh.wait()
        if k + 1 < N_CHUNKS:
            if k - 1 >= 0:
                for h in pend_w.pop(k - 1):
                    h.wait()
            gh = start_gather(k + 1)
        pend_w[k] = start_writebacks(k)
    for kk in sorted(pend_w):
        for h in pend_w[kk]:
            h.wait()


@functools.cache
def _sc_gather():
    return pl.kernel(
        _gather_body,
        out_type=jax.ShapeDtypeStruct((CG, SEG_B, 128), jnp.float32),
        mesh=plsc.VectorSubcoreMesh(core_axis_name="c", subcore_axis_name="s"),
        scratch_types=[
            pltpu.VMEM((IDX_PER_W,), jnp.int32),
            pltpu.VMEM((CHUNK, EDIMP), jnp.float32),
            pltpu.VMEM((CHUNK, EDIMP), jnp.float32),
            pltpu.SemaphoreType.DMA,
            pltpu.SemaphoreType.DMA,
            pltpu.SemaphoreType.DMA,
            pltpu.SemaphoreType.DMA,
        ],
        compiler_params=pltpu.CompilerParams(use_tc_tiling_on_sc=False),
    )


def _mlp_body(x_ref, w1e_ref, w1o_ref, b1_ref, g_ref, bt_ref, w2_ref, b2_ref,
              o_ref):
    h = b1_ref[...]
    for c in range(CG):
        u = lax.bitcast_convert_type(x_ref[c], jnp.int32)
        xe = lax.bitcast_convert_type(u << 16, jnp.float32).astype(jnp.bfloat16)
        xo = lax.bitcast_convert_type(u & jnp.int32(-65536),
                                      jnp.float32).astype(jnp.bfloat16)
        h = h + jnp.dot(xe, w1e_ref[c], preferred_element_type=jnp.float32)
        h = h + jnp.dot(xo, w1o_ref[c], preferred_element_type=jnp.float32)
    mean = jnp.mean(h, axis=1, keepdims=True)
    var = jnp.mean((h - mean) ** 2, axis=1, keepdims=True)
    ln = (h - mean) * lax.rsqrt(var + 1e-5) * g_ref[...] + bt_ref[...]
    a = jnp.where(ln > 0, ln, jnp.exp(ln) - 1.0)
    o_ref[...] = jnp.dot(a, w2_ref[...], preferred_element_type=jnp.float32) + b2_ref[...]


def _mlp(x3, W1e, W1o, b1, gamma, beta, W2, b2, block_b=1024):
    grid = (SEG_B // block_b,)
    return pl.pallas_call(
        _mlp_body,
        grid=grid,
        in_specs=[
            pl.BlockSpec((CG, block_b, 128), lambda i: (0, i, 0)),
            pl.BlockSpec((CG, 128, HDIM), lambda i: (0, 0, 0)),
            pl.BlockSpec((CG, 128, HDIM), lambda i: (0, 0, 0)),
            pl.BlockSpec((1, HDIM), lambda i: (0, 0)),
            pl.BlockSpec((1, HDIM), lambda i: (0, 0)),
            pl.BlockSpec((1, HDIM), lambda i: (0, 0)),
            pl.BlockSpec((HDIM, HDIM), lambda i: (0, 0)),
            pl.BlockSpec((1, HDIM), lambda i: (0, 0)),
        ],
        out_specs=pl.BlockSpec((block_b, HDIM), lambda i: (i, 0)),
        out_shape=jax.ShapeDtypeStruct((SEG_B, HDIM), jnp.float32),
        compiler_params=pltpu.CompilerParams(
            dimension_semantics=("arbitrary",),
        ),
    )(x3, W1e, W1o, b1, gamma, beta, W2, b2)


def kernel(inv_glyphs, emb, W1, b1, gamma, beta, W2, b2):
    emb_p = lax.bitcast_convert_type(
        emb.astype(jnp.bfloat16).reshape(VOCAB, EDIMP, 2), jnp.float32)
    pad_col = (jnp.arange(SEG_B, dtype=jnp.int32) % VOCAB)[:, None]
    g32 = inv_glyphs.astype(jnp.int32)
    W1p = jnp.pad(W1, ((0, SLOT_PAD * EDIM - W1.shape[0]), (0, 0)))
    W1p = W1p.reshape(CG, 128, 2, HDIM).astype(jnp.bfloat16)
    W1e = W1p[:, :, 0, :]
    W1o = W1p[:, :, 1, :]
    b1r, gr, btr, b2r = (v.reshape(1, HDIM) for v in (b1, gamma, beta, b2))
    outs = []
    for s in range(NSEG):
        g_s = lax.slice_in_dim(g32, s * SEG_B, (s + 1) * SEG_B, axis=0)
        idx_s = jnp.concatenate([g_s, pad_col], axis=1)
        idx_s = (idx_s.reshape(BCHUNKS, CB, CG, 8)
                 .transpose(2, 0, 3, 1).reshape(-1))
        x3 = _sc_gather()(idx_s, emb_p)
        outs.append(_mlp(x3, W1e, W1o, b1r, gr, btr, W2, b2r))
    return jnp.concatenate(outs, axis=0)


# speedup vs baseline: 1.5013x; 1.5013x over previous
import functools

import jax
import jax.numpy as jnp
from jax import lax
from jax.experimental import pallas as pl
from jax.experimental.pallas import tpu as pltpu
from jax.experimental.pallas import tpu_sc as plsc

VOCAB = 5977
INV_SLOTS = 55
EDIM = 32
HDIM = 128
BATCH = 16384

NC = 2
NS = 16
NW = NC * NS

SLOT_PAD = 56
EDIMP = EDIM // 2
CG = 7
NSEG = 2
SEG_B = BATCH // NSEG
CB = 256
CHUNK = CB * 8
BCHUNKS = SEG_B // CB
N_CHUNKS = CG * BCHUNKS // NW
IDX_PER_W = N_CHUNKS * CHUNK


def _gather_body(idx_hbm, emb_hbm, out_hbm, idx_v, r0, r1, tab_s,
                 g0, g1, w0, w1):
    wid = lax.axis_index("s") * NC + lax.axis_index("c")
    rows = (r0, r1)
    gsem = (g0, g1)
    wsem = (w0, w1)

    @pl.when(lax.axis_index("s") == 0)
    def _load_table():
        pltpu.sync_copy(emb_hbm, tab_s)

    pltpu.sync_copy(idx_hbm.at[pl.ds(wid * IDX_PER_W, IDX_PER_W)], idx_v)
    plsc.subcore_barrier()

    def start_gather(k):
        return pltpu.async_copy(
            tab_s.at[idx_v.at[pl.ds(k * CHUNK, CHUNK)]],
            rows[k % 2], gsem[k % 2])

    def start_writebacks(k):
        t = wid * N_CHUNKS + k
        c = t // BCHUNKS
        b0 = (t % BCHUNKS) * CB
        return [
            pltpu.async_copy(
                rows[k % 2].at[pl.ds(j * CB, CB), :],
                out_hbm.at[c, pl.ds(b0, CB), pl.ds(EDIMP * j, EDIMP)],
                wsem[k % 2])
            for j in range(8)
        ]

    pend_w = {}
    gh = start_gather(0)
    for k in range(N_CHUNKS):
        gh.wait()
        if k + 1 < N_CHUNKS:
            if k - 1 >= 0:
                for h in pend_w.pop(k - 1):
                    h.wait()
            gh = start_gather(k + 1)
        pend_w[k] = start_writebacks(k)
    for kk in sorted(pend_w):
        for h in pend_w[kk]:
            h.wait()


@functools.cache
def _sc_gather():
    return pl.kernel(
        _gather_body,
        out_type=jax.ShapeDtypeStruct((CG, SEG_B, 128), jnp.float32),
        mesh=plsc.VectorSubcoreMesh(core_axis_name="c", subcore_axis_name="s"),
        scratch_types=[
            pltpu.VMEM((IDX_PER_W,), jnp.int32),
            pltpu.VMEM((CHUNK, EDIMP), jnp.float32),
            pltpu.VMEM((CHUNK, EDIMP), jnp.float32),
            pltpu.VMEM_SHARED((VOCAB, EDIMP), jnp.float32),
            pltpu.SemaphoreType.DMA,
            pltpu.SemaphoreType.DMA,
            pltpu.SemaphoreType.DMA,
            pltpu.SemaphoreType.DMA,
        ],
        compiler_params=pltpu.CompilerParams(use_tc_tiling_on_sc=False),
    )


def _mlp_body(x_ref, w1e_ref, w1o_ref, b1_ref, g_ref, bt_ref, w2_ref, b2_ref,
              o_ref):
    h = b1_ref[...]
    for c in range(CG):
        u = lax.bitcast_convert_type(x_ref[c], jnp.int32)
        xe = lax.bitcast_convert_type(u << 16, jnp.float32).astype(jnp.bfloat16)
        xo = lax.bitcast_convert_type(u & jnp.int32(-65536),
                                      jnp.float32).astype(jnp.bfloat16)
        h = h + jnp.dot(xe, w1e_ref[c], preferred_element_type=jnp.float32)
        h = h + jnp.dot(xo, w1o_ref[c], preferred_element_type=jnp.float32)
    mean = jnp.mean(h, axis=1, keepdims=True)
    var = jnp.mean((h - mean) ** 2, axis=1, keepdims=True)
    ln = (h - mean) * lax.rsqrt(var + 1e-5) * g_ref[...] + bt_ref[...]
    a = jnp.where(ln > 0, ln, jnp.exp(ln) - 1.0)
    o_ref[...] = jnp.dot(a, w2_ref[...], preferred_element_type=jnp.float32) + b2_ref[...]


def _mlp(x3, W1e, W1o, b1, gamma, beta, W2, b2, block_b=1024):
    grid = (SEG_B // block_b,)
    return pl.pallas_call(
        _mlp_body,
        grid=grid,
        in_specs=[
            pl.BlockSpec((CG, block_b, 128), lambda i: (0, i, 0)),
            pl.BlockSpec((CG, 128, HDIM), lambda i: (0, 0, 0)),
            pl.BlockSpec((CG, 128, HDIM), lambda i: (0, 0, 0)),
            pl.BlockSpec((1, HDIM), lambda i: (0, 0)),
            pl.BlockSpec((1, HDIM), lambda i: (0, 0)),
            pl.BlockSpec((1, HDIM), lambda i: (0, 0)),
            pl.BlockSpec((HDIM, HDIM), lambda i: (0, 0)),
            pl.BlockSpec((1, HDIM), lambda i: (0, 0)),
        ],
        out_specs=pl.BlockSpec((block_b, HDIM), lambda i: (i, 0)),
        out_shape=jax.ShapeDtypeStruct((SEG_B, HDIM), jnp.float32),
        compiler_params=pltpu.CompilerParams(
            dimension_semantics=("arbitrary",),
        ),
    )(x3, W1e, W1o, b1, gamma, beta, W2, b2)


def kernel(inv_glyphs, emb, W1, b1, gamma, beta, W2, b2):
    emb_p = lax.bitcast_convert_type(
        emb.astype(jnp.bfloat16).reshape(VOCAB, EDIMP, 2), jnp.float32)
    pad_col = (jnp.arange(SEG_B, dtype=jnp.int32) % VOCAB)[:, None]
    g32 = inv_glyphs.astype(jnp.int32)
    W1p = jnp.pad(W1, ((0, SLOT_PAD * EDIM - W1.shape[0]), (0, 0)))
    W1p = W1p.reshape(CG, 128, 2, HDIM).astype(jnp.bfloat16)
    W1e = W1p[:, :, 0, :]
    W1o = W1p[:, :, 1, :]
    b1r, gr, btr, b2r = (v.reshape(1, HDIM) for v in (b1, gamma, beta, b2))
    outs = []
    for s in range(NSEG):
        g_s = lax.slice_in_dim(g32, s * SEG_B, (s + 1) * SEG_B, axis=0)
        idx_s = jnp.concatenate([g_s, pad_col], axis=1)
        idx_s = (idx_s.reshape(BCHUNKS, CB, CG, 8)
                 .transpose(2, 0, 3, 1).reshape(-1))
        x3 = _sc_gather()(idx_s, emb_p)
        outs.append(_mlp(x3, W1e, W1o, b1r, gr, btr, W2, b2r))
    return jnp.concatenate(outs, axis=0)
